# Initial kernel scaffold; baseline (speedup 1.0000x reference)
import jax
import jax.numpy as jnp
from jax.experimental import pallas as pl


def kernel(features, coords, W1, W2, batch_size):
    raise NotImplementedError("write your pallas kernel here")


# speedup vs baseline: 35.7201x; 35.7201x over previous
import jax
import jax.numpy as jnp
from jax.experimental import pallas as pl

_DIM = 128
_BLOCK_ROWS = 4096


def _mm2_kernel(f_ref, w1_ref, w2_ref, o_ref):
    t = jnp.dot(f_ref[...], w1_ref[...], preferred_element_type=jnp.float32)
    o_ref[...] = jnp.dot(t, w2_ref[...], preferred_element_type=jnp.float32)


def kernel(features, coords, W1, W2, batch_size):
    del coords, batch_size
    n, d = features.shape
    return pl.pallas_call(
        _mm2_kernel,
        grid=(pl.cdiv(n, _BLOCK_ROWS),),
        in_specs=[
            pl.BlockSpec((_BLOCK_ROWS, d), lambda i: (i, 0)),
            pl.BlockSpec((d, d), lambda i: (0, 0)),
            pl.BlockSpec((d, d), lambda i: (0, 0)),
        ],
        out_specs=pl.BlockSpec((_BLOCK_ROWS, d), lambda i: (i, 0)),
        out_shape=jax.ShapeDtypeStruct((n, d), features.dtype),
    )(features, W1, W2)


# speedup vs baseline: 42.5212x; 1.1904x over previous
import jax
import jax.numpy as jnp
from jax.experimental import pallas as pl

_DIM = 128
_BLOCK_ROWS = 8192


def _mm2_kernel(f_ref, w1_ref, w2_ref, o_ref):
    t = jnp.dot(f_ref[...], w1_ref[...], preferred_element_type=jnp.float32)
    o_ref[...] = jnp.dot(t, w2_ref[...], preferred_element_type=jnp.float32)


def kernel(features, coords, W1, W2, batch_size):
    del coords, batch_size
    n, d = features.shape
    return pl.pallas_call(
        _mm2_kernel,
        grid=(pl.cdiv(n, _BLOCK_ROWS),),
        in_specs=[
            pl.BlockSpec((_BLOCK_ROWS, d), lambda i: (i, 0)),
            pl.BlockSpec((d, d), lambda i: (0, 0)),
            pl.BlockSpec((d, d), lambda i: (0, 0)),
        ],
        out_specs=pl.BlockSpec((_BLOCK_ROWS, d), lambda i: (i, 0)),
        out_shape=jax.ShapeDtypeStruct((n, d), features.dtype),
    )(features, W1, W2)


# speedup vs baseline: 43.8660x; 1.0316x over previous
import jax
import jax.numpy as jnp
from jax.experimental import pallas as pl

_DIM = 128
_BLOCK_ROWS = 16384


def _mm2_kernel(f_ref, w1_ref, w2_ref, o_ref):
    t = jnp.dot(f_ref[...], w1_ref[...], preferred_element_type=jnp.float32)
    o_ref[...] = jnp.dot(t, w2_ref[...], preferred_element_type=jnp.float32)


def kernel(features, coords, W1, W2, batch_size):
    del coords, batch_size
    n, d = features.shape
    return pl.pallas_call(
        _mm2_kernel,
        grid=(pl.cdiv(n, _BLOCK_ROWS),),
        in_specs=[
            pl.BlockSpec((_BLOCK_ROWS, d), lambda i: (i, 0)),
            pl.BlockSpec((d, d), lambda i: (0, 0)),
            pl.BlockSpec((d, d), lambda i: (0, 0)),
        ],
        out_specs=pl.BlockSpec((_BLOCK_ROWS, d), lambda i: (i, 0)),
        out_shape=jax.ShapeDtypeStruct((n, d), features.dtype),
    )(features, W1, W2)


# speedup vs baseline: 44.1381x; 1.0062x over previous
import jax
import jax.numpy as jnp
from jax.experimental import pallas as pl

_DIM = 128
_BLOCK_ROWS = 24576


def _mm2_kernel(f_ref, w1_ref, w2_ref, o_ref):
    t = jnp.dot(f_ref[...], w1_ref[...], preferred_element_type=jnp.float32)
    o_ref[...] = jnp.dot(t, w2_ref[...], preferred_element_type=jnp.float32)


def kernel(features, coords, W1, W2, batch_size):
    del coords, batch_size
    n, d = features.shape
    return pl.pallas_call(
        _mm2_kernel,
        grid=(pl.cdiv(n, _BLOCK_ROWS),),
        in_specs=[
            pl.BlockSpec((_BLOCK_ROWS, d), lambda i: (i, 0)),
            pl.BlockSpec((d, d), lambda i: (0, 0)),
            pl.BlockSpec((d, d), lambda i: (0, 0)),
        ],
        out_specs=pl.BlockSpec((_BLOCK_ROWS, d), lambda i: (i, 0)),
        out_shape=jax.ShapeDtypeStruct((n, d), features.dtype),
    )(features, W1, W2)
